# baseline (speedup 1.0000x reference)
import functools

import jax
import jax.numpy as jnp
from jax import lax
from jax.experimental import pallas as pl
from jax.experimental.pallas import tpu as pltpu
from jax.experimental.pallas import tpu_sc as plsc

N_NODES = 10000
N_EDGES = 320000
D = 128

NC = 2
NS = 16
NW = NC * NS
C = 40
NCHUNKS = N_EDGES // C
K_PER_W = NCHUNKS // NW
NP = 10112
ROWS_PER_TILE = NP // NS


def _sc_body(x_hbm, src_hbm, dst_hbm, attr_hbm, z128_hbm, z16_hbm,
             ld_hbm, psum_hbm, pcnt_hbm,
             idx_sA, idx_sB, idx_dA, idx_dB, wbA, wbB, ones_v,
             bufsA, bufsB, bufdA, bufdB, sums_sh, cnts_sh, semA, semB):
    cid = lax.axis_index("c")
    sid = lax.axis_index("s")
    wid = sid * NC + cid

    def _init_ones(i, carry):
        ones_v[i, :] = jnp.ones((16,), jnp.float32)
        return carry
    lax.fori_loop(0, C, _init_ones, 0)

    row0 = sid * ROWS_PER_TILE
    pltpu.sync_copy(z128_hbm, sums_sh.at[pl.ds(row0, ROWS_PER_TILE)])
    pltpu.sync_copy(z16_hbm, cnts_sh.at[pl.ds(row0, ROWS_PER_TILE)])
    plsc.subcore_barrier()

    def _fire(k, idx_s, idx_d, wb, bs, bd, sem):
        base = (wid + NW * k) * C
        pltpu.sync_copy(src_hbm.at[pl.ds(base, C)], idx_s)
        pltpu.sync_copy(dst_hbm.at[pl.ds(base, C)], idx_d)
        pltpu.sync_copy(attr_hbm.at[pl.ds(base, C)], wb)
        pltpu.async_copy(x_hbm.at[idx_s], bs, sem)
        pltpu.async_copy(x_hbm.at[idx_d], bd, sem)

    def _waitg(idx_s, idx_d, bs, bd, sem):
        pltpu.make_async_copy(x_hbm.at[idx_s], bs, sem).wait()
        pltpu.make_async_copy(x_hbm.at[idx_d], bd, sem).wait()

    def _compute_store(k, idx_d, wb, bs, bd):
        for g in range((C + 15) // 16):
            off = min(g * 16, C - 16)
            inv16 = 1.0 / wb[pl.ds(off, 16)]
            for r in range(g * 16, min((g + 1) * 16, C)):
                inv = inv16[r - off]
                for j in range(D // 16):
                    sl = pl.ds(j * 16, 16)
                    bd[r, sl] = (bd[r, sl] - bs[r, sl]) * inv
        base = (wid + NW * k) * C
        pltpu.sync_copy(bd, ld_hbm.at[pl.ds(base, C)])
        pltpu.sync_copy(bd, sums_sh.at[idx_d], add=True)
        pltpu.sync_copy(ones_v, cnts_sh.at[idx_d], add=True)

    A = (idx_sA, idx_dA, wbA, bufsA, bufdA, semA)
    B = (idx_sB, idx_dB, wbB, bufsB, bufdB, semB)

    def _wait_comp(k, s):
        idx_s, idx_d, wb, bs, bd, sem = s
        _waitg(idx_s, idx_d, bs, bd, sem)
        _compute_store(k, idx_d, wb, bs, bd)

    _fire(0, *A)

    def _body2(g, carry):
        k0 = 2 * g
        _fire(k0 + 1, *B)
        _wait_comp(k0, A)
        _fire(k0 + 2, *A)
        _wait_comp(k0 + 1, B)
        return carry
    lax.fori_loop(0, (K_PER_W - 2) // 2, _body2, 0)

    _fire(K_PER_W - 1, *B)
    _wait_comp(K_PER_W - 2, A)
    _wait_comp(K_PER_W - 1, B)

    plsc.subcore_barrier()
    pltpu.sync_copy(sums_sh.at[pl.ds(row0, ROWS_PER_TILE)],
                    psum_hbm.at[cid, pl.ds(row0, ROWS_PER_TILE)])
    pltpu.sync_copy(cnts_sh.at[pl.ds(row0, ROWS_PER_TILE)],
                    pcnt_hbm.at[cid, pl.ds(row0, ROWS_PER_TILE)])


_sc_kernel = functools.partial(
    pl.kernel,
    compiler_params=pltpu.CompilerParams(use_tc_tiling_on_sc=False),
    out_type=(
        jax.ShapeDtypeStruct((N_EDGES, D), jnp.float32),
        jax.ShapeDtypeStruct((NC, NP, D), jnp.float32),
        jax.ShapeDtypeStruct((NC, NP, 16), jnp.float32),
    ),
    mesh=plsc.VectorSubcoreMesh(core_axis_name="c", subcore_axis_name="s"),
    scratch_types=[
        pltpu.VMEM((C,), jnp.int32),
        pltpu.VMEM((C,), jnp.int32),
        pltpu.VMEM((C,), jnp.int32),
        pltpu.VMEM((C,), jnp.int32),
        pltpu.VMEM((C,), jnp.float32),
        pltpu.VMEM((C,), jnp.float32),
        pltpu.VMEM((C, 16), jnp.float32),
        pltpu.VMEM((C, D), jnp.float32),
        pltpu.VMEM((C, D), jnp.float32),
        pltpu.VMEM((C, D), jnp.float32),
        pltpu.VMEM((C, D), jnp.float32),
        pltpu.VMEM_SHARED((NP, D), jnp.float32),
        pltpu.VMEM_SHARED((NP, 16), jnp.float32),
        pltpu.SemaphoreType.DMA,
        pltpu.SemaphoreType.DMA,
    ],
)(_sc_body)


def _combine_body(ps_ref, pc_ref, out_ref):
    s = ps_ref[0] + ps_ref[1]
    c = pc_ref[0, :, 0:1] + pc_ref[1, :, 0:1]
    out_ref[...] = s / jnp.maximum(c, 1.0)


_NB = 1000


def _combine(psum, pcnt):
    return pl.pallas_call(
        _combine_body,
        grid=(N_NODES // _NB,),
        in_specs=[
            pl.BlockSpec((NC, _NB, D), lambda i: (0, i, 0)),
            pl.BlockSpec((NC, _NB, 16), lambda i: (0, i, 0)),
        ],
        out_specs=pl.BlockSpec((_NB, D), lambda i: (i, 0)),
        out_shape=jax.ShapeDtypeStruct((N_NODES, D), jnp.float32),
    )(psum, pcnt)


def kernel(x, edge_index, edge_attr):
    src = edge_index[0]
    dst = edge_index[1]
    attr = edge_attr.reshape(N_EDGES)
    z128 = jnp.zeros((ROWS_PER_TILE, D), jnp.float32)
    z16 = jnp.zeros((ROWS_PER_TILE, 16), jnp.float32)
    ld, psum, pcnt = _sc_kernel(x, src, dst, attr, z128, z16)
    node = _combine(psum, pcnt)
    return node, ld


# speedup vs baseline: 1.0057x; 1.0057x over previous
import functools

import jax
import jax.numpy as jnp
from jax import lax
from jax.experimental import pallas as pl
from jax.experimental.pallas import tpu as pltpu
from jax.experimental.pallas import tpu_sc as plsc

N_NODES = 10000
N_EDGES = 320000
D = 128

NC = 2
NS = 16
NW = NC * NS
C = 80
NCHUNKS = N_EDGES // C
K_PER_W = NCHUNKS // NW
NP = 10112
ROWS_PER_TILE = NP // NS


def _sc_body(x_hbm, src_hbm, dst_hbm, attr_hbm, z128_hbm, z16_hbm,
             ld_hbm, psum_hbm, pcnt_hbm,
             idx_s, idx_d, wbuf, ones_v, bufs, bufd, sums_sh, cnts_sh, sem):
    cid = lax.axis_index("c")
    sid = lax.axis_index("s")
    wid = sid * NC + cid

    def _init_ones(i, carry):
        ones_v[i, :] = jnp.ones((16,), jnp.float32)
        return carry
    lax.fori_loop(0, C, _init_ones, 0)

    row0 = sid * ROWS_PER_TILE
    pltpu.sync_copy(z128_hbm, sums_sh.at[pl.ds(row0, ROWS_PER_TILE)])
    pltpu.sync_copy(z16_hbm, cnts_sh.at[pl.ds(row0, ROWS_PER_TILE)])
    plsc.subcore_barrier()

    def _chunk(k, carry):
        c = wid + NW * k
        base = c * C
        pltpu.sync_copy(src_hbm.at[pl.ds(base, C)], idx_s)
        pltpu.sync_copy(dst_hbm.at[pl.ds(base, C)], idx_d)
        pltpu.sync_copy(attr_hbm.at[pl.ds(base, C)], wbuf)
        pltpu.async_copy(x_hbm.at[idx_s], bufs, sem).wait()
        pltpu.async_copy(x_hbm.at[idx_d], bufd, sem).wait()

        def _grp(g, rc):
            inv16 = 1.0 / wbuf[pl.ds(g * 16, 16)]
            for r16 in range(16):
                r = g * 16 + r16
                inv = inv16[r16]
                for j in range(D // 16):
                    sl = pl.ds(j * 16, 16)
                    bufd[r, sl] = (bufd[r, sl] - bufs[r, sl]) * inv
            return rc
        lax.fori_loop(0, C // 16, _grp, 0)

        pltpu.sync_copy(bufd, ld_hbm.at[pl.ds(base, C)])
        pltpu.sync_copy(bufd, sums_sh.at[idx_d], add=True)
        pltpu.sync_copy(ones_v, cnts_sh.at[idx_d], add=True)
        return carry
    lax.fori_loop(0, K_PER_W, _chunk, 0)

    plsc.subcore_barrier()
    pltpu.sync_copy(sums_sh.at[pl.ds(row0, ROWS_PER_TILE)],
                    psum_hbm.at[cid, pl.ds(row0, ROWS_PER_TILE)])
    pltpu.sync_copy(cnts_sh.at[pl.ds(row0, ROWS_PER_TILE)],
                    pcnt_hbm.at[cid, pl.ds(row0, ROWS_PER_TILE)])


_sc_kernel = functools.partial(
    pl.kernel,
    compiler_params=pltpu.CompilerParams(use_tc_tiling_on_sc=False),
    out_type=(
        jax.ShapeDtypeStruct((N_EDGES, D), jnp.float32),
        jax.ShapeDtypeStruct((NC, NP, D), jnp.float32),
        jax.ShapeDtypeStruct((NC, NP, 16), jnp.float32),
    ),
    mesh=plsc.VectorSubcoreMesh(core_axis_name="c", subcore_axis_name="s"),
    scratch_types=[
        pltpu.VMEM((C,), jnp.int32),
        pltpu.VMEM((C,), jnp.int32),
        pltpu.VMEM((C,), jnp.float32),
        pltpu.VMEM((C, 16), jnp.float32),
        pltpu.VMEM((C, D), jnp.float32),
        pltpu.VMEM((C, D), jnp.float32),
        pltpu.VMEM_SHARED((NP, D), jnp.float32),
        pltpu.VMEM_SHARED((NP, 16), jnp.float32),
        pltpu.SemaphoreType.DMA,
    ],
)(_sc_body)


def _combine_body(ps_ref, pc_ref, out_ref):
    s = ps_ref[0] + ps_ref[1]
    c = pc_ref[0, :, 0:1] + pc_ref[1, :, 0:1]
    out_ref[...] = s / jnp.maximum(c, 1.0)


_NB = 1000


def _combine(psum, pcnt):
    return pl.pallas_call(
        _combine_body,
        grid=(N_NODES // _NB,),
        in_specs=[
            pl.BlockSpec((NC, _NB, D), lambda i: (0, i, 0)),
            pl.BlockSpec((NC, _NB, 16), lambda i: (0, i, 0)),
        ],
        out_specs=pl.BlockSpec((_NB, D), lambda i: (i, 0)),
        out_shape=jax.ShapeDtypeStruct((N_NODES, D), jnp.float32),
    )(psum, pcnt)


def kernel(x, edge_index, edge_attr):
    src = edge_index[0]
    dst = edge_index[1]
    attr = edge_attr.reshape(N_EDGES)
    z128 = jnp.zeros((ROWS_PER_TILE, D), jnp.float32)
    z16 = jnp.zeros((ROWS_PER_TILE, 16), jnp.float32)
    ld, psum, pcnt = _sc_kernel(x, src, dst, attr, z128, z16)
    node = _combine(psum, pcnt)
    return node, ld
